# Initial kernel scaffold; baseline (speedup 1.0000x reference)
import jax
import jax.numpy as jnp
from jax.experimental import pallas as pl
from jax.experimental.pallas import tpu as pltpu

B = 128
H = 128
NOISE = 128
T = 10
NPT = 8192
NC = 4
EMB = 32
NN = 3 * NPT
TAU = 3.0
LOGN = 13


def _dot(a, b):
    return jax.lax.dot_general(a, b, (((1,), (0,)), ((), ())),
                               preferred_element_type=jnp.float32)


def _dist_block(emb, proj):
    chunks = []
    CH = 1024
    for nb in range(0, NPT, CH):
        diff = emb[nb:nb + CH, :][None, :, :] - proj[:, None, :]
        chunks.append(jnp.sqrt(jnp.sum(diff * diff, axis=-1)))
    return jnp.concatenate(chunks, axis=1)


def _roll(x, shift):
    return pltpu.roll(x, shift, axis=1)


def _bitonic_sort_rows(x):
    iota = jax.lax.broadcasted_iota(jnp.int32, (1, NPT), 1)
    for s in range(1, LOGN + 1):
        k = 1 << s
        for j_log in range(s - 1, -1, -1):
            j = 1 << j_log
            bit_clear = (iota & j) == 0
            asc = (iota & k) == 0
            if k == NPT:
                cond = bit_clear
            else:
                cond = asc == bit_clear
            partner = jnp.where(bit_clear, _roll(x, -j), _roll(x, j))
            lo = jnp.minimum(x, partner)
            hi = jnp.maximum(x, partner)
            x = jnp.where(cond, lo, hi)
    return x


def _cumsum_rows(x):
    iota = jax.lax.broadcasted_iota(jnp.int32, (1, NPT), 1)
    for s in range(LOGN):
        sh = 1 << s
        x = x + jnp.where(iota >= sh, _roll(x, sh), 0)
    return x


def _step_body(z_ref, wi1_ref, wi2c_ref, wi2h_ref, wih_ref, whh_ref,
               wtype_ref, wn_ref, wdt_ref, wdn_ref, emb_ref, gt_ref, gk_ref,
               x_out_ref, node_out_ref,
               h_ref, c_ref, inp_ref, x_scr, cand_scr):
    i = pl.program_id(0)
    j = pl.program_id(1)

    @pl.when((j == 0) & (i == 0))
    def _init():
        inter = jnp.tanh(_dot(z_ref[...], wi1_ref[...]))
        c_ref[...] = jnp.tanh(_dot(inter, wi2c_ref[...]))
        h_ref[...] = jnp.tanh(_dot(inter, wi2h_ref[...]))
        inp_ref[...] = jnp.zeros((B, H), jnp.float32)

    @pl.when(j == 0)
    def _lstm_and_type():
        h = h_ref[...]
        c = c_ref[...]
        gates = _dot(inp_ref[...], wih_ref[...]) + _dot(h, whh_ref[...])
        gi = gates[:, 0 * H:1 * H]
        gf = gates[:, 1 * H:2 * H]
        gg = gates[:, 2 * H:3 * H]
        go = gates[:, 3 * H:4 * H]
        c = jax.nn.sigmoid(gf) * c + jax.nn.sigmoid(gi) * jnp.tanh(gg)
        h = jax.nn.sigmoid(go) * jnp.tanh(c)
        c_ref[...] = c
        h_ref[...] = h
        logits = _dot(h, wtype_ref[...])[:, :NC]
        scores = logits + gt_ref[0]
        x = jnp.argmax(scores, axis=1).astype(jnp.int32)[:, None]
        x_scr[...] = x
        x_out_ref[0] = x

    proj = _dot(h_ref[...], wn_ref[j])
    dist = _dist_block(emb_ref[j], proj)
    vals = _bitonic_sort_rows(dist)
    scores_r = gk_ref[0, 0] - vals
    pos = jnp.argmax(scores_r, axis=1).astype(jnp.int32)[:, None]
    iota = jax.lax.broadcasted_iota(jnp.int32, (1, NPT), 1)
    vstar = jnp.sum(jnp.where(iota == pos, vals, 0.0), axis=1)[:, None]
    c1 = jnp.sum((dist < vstar).astype(jnp.int32), axis=1)[:, None]
    k = pos - c1
    eq = dist == vstar
    occ = _cumsum_rows(eq.astype(jnp.int32))
    hit = jnp.logical_and(occ == (k + 1), eq)
    cand = jnp.argmax(hit, axis=1).astype(jnp.int32)[:, None]
    cand_scr[:, pl.ds(j, 1)] = cand + j * NPT

    @pl.when(j == 2)
    def _finalize():
        x = x_scr[...]
        xc = jnp.clip(x, 0, 2)
        n0 = cand_scr[:, 0:1]
        n1 = cand_scr[:, 1:2]
        n2 = cand_scr[:, 2:3]
        sel = jnp.where(xc == 0, n0, jnp.where(xc == 1, n1, n2))
        chosen = jnp.where(x == 3, NN, sel)
        node_out_ref[0] = chosen

        @pl.when(i < T - 1)
        def _next_inputs():
            oh_t = (jax.lax.broadcasted_iota(jnp.int32, (B, 8), 1)
                    == x).astype(jnp.float32)
            acc = _dot(oh_t, wdt_ref[...])
            CH = 2048
            for nb in range(0, NN + CH, CH):
                oh_n = (jax.lax.broadcasted_iota(jnp.int32, (B, CH), 1) + nb
                        == chosen).astype(jnp.float32)
                acc = acc + _dot(oh_n, wdn_ref[nb:nb + CH, :])
            inp_ref[...] = acc


def _onehot_body(x_ref, node_ref, type_out_ref, node_out_ref):
    x = x_ref[0]
    node = node_ref[0]
    iota4 = jax.lax.broadcasted_iota(jnp.int32, (B, NC), 1)
    type_out_ref[:, 0, :] = (iota4 == x).astype(jnp.float32)
    iotan = jax.lax.broadcasted_iota(jnp.int32, (B, NN + 1), 1)
    node_out_ref[:, 0, :] = (iotan == node).astype(jnp.float32)


def kernel(z, params):
    p = params
    f32 = jnp.float32

    wi1 = p['W_init1'].T
    wi2c = p['W_init2c'].T
    wi2h = p['W_init2h'].T
    wih = p['W_ih'].T
    whh = p['W_hh'].T
    wtype = jnp.zeros((H, 128), f32).at[:, :NC].set(p['W_type'].T)
    wn = jnp.stack([p['W_n0'], p['W_n1'], p['W_n2']], 0).transpose(0, 2, 1)
    wdt = jnp.zeros((8, H), f32).at[:NC, :].set(p['W_dt'].T)
    wdn = jnp.zeros((NN + 2 * 2048, H), f32).at[:NN + 1, :].set(p['W_dn'].T)
    embs = p['node_embs']

    base = jax.random.key(42)
    gt = jnp.stack([jax.random.gumbel(jax.random.fold_in(base, 100 * i),
                                      (B, NC), f32) for i in range(T)], 0)
    gk = jnp.stack([
        jnp.stack([jax.random.gumbel(jax.random.fold_in(base, 100 * i + 1 + t),
                                     (B, NPT), f32) for t in range(3)], 0)
        for i in range(T)], 0)

    grid = (T, 3)
    const2 = lambda i, j: (0, 0)
    const3 = lambda i, j: (0, 0, 0)
    step_i3 = lambda i, j: (i, 0, 0)

    x_ids, node_ids = pl.pallas_call(
        _step_body,
        grid=grid,
        in_specs=[
            pl.BlockSpec((B, NOISE), const2),
            pl.BlockSpec((NOISE, H), const2),
            pl.BlockSpec((H, H), const2),
            pl.BlockSpec((H, H), const2),
            pl.BlockSpec((H, 4 * H), const2),
            pl.BlockSpec((H, 4 * H), const2),
            pl.BlockSpec((H, 128), const2),
            pl.BlockSpec((3, H, EMB), const3),
            pl.BlockSpec((8, H), const2),
            pl.BlockSpec((NN + 2 * 2048, H), const2),
            pl.BlockSpec((3, NPT, EMB), const3),
            pl.BlockSpec((1, B, NC), step_i3),
            pl.BlockSpec((1, 1, B, NPT), lambda i, j: (i, j, 0, 0)),
        ],
        out_specs=[
            pl.BlockSpec((1, B, 1), step_i3),
            pl.BlockSpec((1, B, 1), step_i3),
        ],
        out_shape=[
            jax.ShapeDtypeStruct((T, B, 1), jnp.int32),
            jax.ShapeDtypeStruct((T, B, 1), jnp.int32),
        ],
        scratch_shapes=[
            pltpu.VMEM((B, H), f32),
            pltpu.VMEM((B, H), f32),
            pltpu.VMEM((B, H), f32),
            pltpu.VMEM((B, 1), jnp.int32),
            pltpu.VMEM((B, 8), jnp.int32),
        ],
        compiler_params=pltpu.CompilerParams(
            dimension_semantics=("arbitrary", "arbitrary")),
    )(z, wi1, wi2c, wi2h, wih, whh, wtype, wn, wdt, wdn, embs, gt, gk)

    out_type, out_node = pl.pallas_call(
        _onehot_body,
        grid=(T,),
        in_specs=[
            pl.BlockSpec((1, B, 1), lambda i: (i, 0, 0)),
            pl.BlockSpec((1, B, 1), lambda i: (i, 0, 0)),
        ],
        out_specs=[
            pl.BlockSpec((B, 1, NC), lambda i: (0, i, 0)),
            pl.BlockSpec((B, 1, NN + 1), lambda i: (0, i, 0)),
        ],
        out_shape=[
            jax.ShapeDtypeStruct((B, T, NC), f32),
            jax.ShapeDtypeStruct((B, T, NN + 1), f32),
        ],
        compiler_params=pltpu.CompilerParams(
            dimension_semantics=("arbitrary",)),
    )(x_ids, node_ids)
    return out_type, out_node


# speedup vs baseline: 2.3050x; 2.3050x over previous
import jax
import jax.numpy as jnp
from jax.experimental import pallas as pl
from jax.experimental.pallas import tpu as pltpu

B = 128
H = 128
NOISE = 128
T = 10
NPT = 8192
NC = 4
EMB = 32
NN = 3 * NPT
TAU = 3.0
LOGN = 13


def _dot(a, b):
    return jax.lax.dot_general(a, b, (((1,), (0,)), ((), ())),
                               preferred_element_type=jnp.float32)


def _dist_into(dist_ref, embT, proj):
    CH = 1024
    for nb in range(0, NPT, CH):
        def term(k):
            d = embT[k:k + 1, nb:nb + CH] - proj[:, k:k + 1]
            return d * d
        acc = None
        for g in range(EMB // 8):
            t = [term(8 * g + s) for s in range(8)]
            sg = (((t[0] + t[4]) + (t[2] + t[6]))
                  + ((t[1] + t[5]) + (t[3] + t[7])))
            acc = sg if acc is None else acc + sg
        dist_ref[:, nb:nb + CH] = jnp.sqrt(acc)


def _roll(x, shift):
    return pltpu.roll(x, shift % NPT, axis=1)


def _bitonic_sort_rows(vals_ref):
    iota = jax.lax.broadcasted_iota(jnp.int32, (1, NPT), 1)
    for s in range(1, LOGN + 1):
        k = 1 << s
        for j_log in range(s - 1, -1, -1):
            j = 1 << j_log
            bit_clear = (iota & j) == 0
            asc = (iota & k) == 0
            if k == NPT:
                cond = bit_clear
            else:
                cond = asc == bit_clear
            x = vals_ref[...]
            partner = jnp.where(bit_clear, _roll(x, -j), _roll(x, j))
            lo = jnp.minimum(x, partner)
            hi = jnp.maximum(x, partner)
            vals_ref[...] = jnp.where(cond, lo, hi)


def _cumsum_rows(x):
    iota = jax.lax.broadcasted_iota(jnp.int32, (1, NPT), 1)
    for s in range(LOGN):
        sh = 1 << s
        x = x + jnp.where(iota >= sh, _roll(x, sh), 0)
    return x


def _step_body(z_ref, wi1_ref, wi2c_ref, wi2h_ref, wih_ref, whh_ref,
               wtype_ref, wn_ref, wdt_ref, wdn_ref, emb_ref, gt_ref, gk_ref,
               x_out_ref, node_out_ref,
               h_ref, c_ref, inp_ref, x_scr, cand0_scr, cand1_scr,
               dist_scr, vals_scr):
    i = pl.program_id(0)
    j = pl.program_id(1)

    @pl.when((j == 0) & (i == 0))
    def _init():
        inter = jnp.tanh(_dot(z_ref[...], wi1_ref[...]))
        c_ref[...] = jnp.tanh(_dot(inter, wi2c_ref[...]))
        h_ref[...] = jnp.tanh(_dot(inter, wi2h_ref[...]))
        inp_ref[...] = jnp.zeros((B, H), jnp.float32)

    @pl.when(j == 0)
    def _lstm_and_type():
        h = h_ref[...]
        c = c_ref[...]
        gates = _dot(inp_ref[...], wih_ref[...]) + _dot(h, whh_ref[...])
        gi = gates[:, 0 * H:1 * H]
        gf = gates[:, 1 * H:2 * H]
        gg = gates[:, 2 * H:3 * H]
        go = gates[:, 3 * H:4 * H]
        c = jax.nn.sigmoid(gf) * c + jax.nn.sigmoid(gi) * jnp.tanh(gg)
        h = jax.nn.sigmoid(go) * jnp.tanh(c)
        c_ref[...] = c
        h_ref[...] = h
        logits = _dot(h, wtype_ref[...])[:, :NC]
        scores = logits + gt_ref[0]
        x = jnp.argmax(scores, axis=1).astype(jnp.int32)[:, None]
        x_scr[...] = x
        x_out_ref[0] = x

    proj = _dot(h_ref[...], wn_ref[j])
    _dist_into(dist_scr, emb_ref[j], proj)
    dist = dist_scr[...]
    vals_scr[...] = dist
    _bitonic_sort_rows(vals_scr)
    vals = vals_scr[...]
    scores_r = gk_ref[0, 0] - vals
    pos = jnp.argmax(scores_r, axis=1).astype(jnp.int32)[:, None]
    iota = jax.lax.broadcasted_iota(jnp.int32, (1, NPT), 1)
    vstar = jnp.sum(jnp.where(iota == pos, vals, 0.0), axis=1)[:, None]
    c1 = jnp.sum((dist < vstar).astype(jnp.int32), axis=1)[:, None]
    k = pos - c1
    eq = dist == vstar
    occ = _cumsum_rows(eq.astype(jnp.int32))
    hit = jnp.logical_and(occ == (k + 1), eq)
    cand = jnp.argmax(hit.astype(jnp.float32),
                      axis=1).astype(jnp.int32)[:, None]
    gcand = cand + j * NPT

    @pl.when(j == 0)
    def _st0():
        cand0_scr[...] = gcand

    @pl.when(j == 1)
    def _st1():
        cand1_scr[...] = gcand

    @pl.when(j == 2)
    def _finalize():
        x = x_scr[...]
        xc = jnp.clip(x, 0, 2)
        n0 = cand0_scr[...]
        n1 = cand1_scr[...]
        n2 = gcand
        sel = jnp.where(xc == 0, n0, jnp.where(xc == 1, n1, n2))
        chosen = jnp.where(x == 3, NN, sel)
        node_out_ref[0] = chosen

        @pl.when(i < T - 1)
        def _next_inputs():
            oh_t = (jax.lax.broadcasted_iota(jnp.int32, (B, 8), 1)
                    == x).astype(jnp.float32)
            acc = _dot(oh_t, wdt_ref[...])
            CH = 2048
            for nb in range(0, NN + CH, CH):
                oh_n = (jax.lax.broadcasted_iota(jnp.int32, (B, CH), 1) + nb
                        == chosen).astype(jnp.float32)
                acc = acc + _dot(oh_n, wdn_ref[nb:nb + CH, :])
            inp_ref[...] = acc


def _onehot_body(x_ref, node_ref, type_out_ref, node_out_ref):
    x = x_ref[0]
    node = node_ref[0]
    iota4 = jax.lax.broadcasted_iota(jnp.int32, (B, NC), 1)
    type_out_ref[0] = (iota4 == x).astype(jnp.float32)
    iotan = jax.lax.broadcasted_iota(jnp.int32, (B, NN + 1), 1)
    node_out_ref[0] = (iotan == node).astype(jnp.float32)


def kernel(z, params):
    p = params
    f32 = jnp.float32

    wi1 = p['W_init1'].T
    wi2c = p['W_init2c'].T
    wi2h = p['W_init2h'].T
    wih = p['W_ih'].T
    whh = p['W_hh'].T
    wtype = jnp.zeros((H, 128), f32).at[:, :NC].set(p['W_type'].T)
    wn = jnp.stack([p['W_n0'], p['W_n1'], p['W_n2']], 0).transpose(0, 2, 1)
    wdt = jnp.zeros((8, H), f32).at[:NC, :].set(p['W_dt'].T)
    wdn = jnp.zeros((NN + 2 * 2048, H), f32).at[:NN + 1, :].set(p['W_dn'].T)
    embs = p['node_embs'].transpose(0, 2, 1)

    base = jax.random.key(42)
    gt = jnp.stack([jax.random.gumbel(jax.random.fold_in(base, 100 * i),
                                      (B, NC), f32) for i in range(T)], 0)
    gk = jnp.stack([
        jnp.stack([jax.random.gumbel(jax.random.fold_in(base, 100 * i + 1 + t),
                                     (B, NPT), f32) for t in range(3)], 0)
        for i in range(T)], 0)

    grid = (T, 3)
    const2 = lambda i, j: (0, 0)
    const3 = lambda i, j: (0, 0, 0)
    step_i3 = lambda i, j: (i, 0, 0)

    x_ids, node_ids = pl.pallas_call(
        _step_body,
        grid=grid,
        in_specs=[
            pl.BlockSpec((B, NOISE), const2),
            pl.BlockSpec((NOISE, H), const2),
            pl.BlockSpec((H, H), const2),
            pl.BlockSpec((H, H), const2),
            pl.BlockSpec((H, 4 * H), const2),
            pl.BlockSpec((H, 4 * H), const2),
            pl.BlockSpec((H, 128), const2),
            pl.BlockSpec((3, H, EMB), const3),
            pl.BlockSpec((8, H), const2),
            pl.BlockSpec((NN + 2 * 2048, H), const2),
            pl.BlockSpec((3, EMB, NPT), const3),
            pl.BlockSpec((1, B, NC), step_i3),
            pl.BlockSpec((1, 1, B, NPT), lambda i, j: (i, j, 0, 0)),
        ],
        out_specs=[
            pl.BlockSpec((1, B, 1), step_i3),
            pl.BlockSpec((1, B, 1), step_i3),
        ],
        out_shape=[
            jax.ShapeDtypeStruct((T, B, 1), jnp.int32),
            jax.ShapeDtypeStruct((T, B, 1), jnp.int32),
        ],
        scratch_shapes=[
            pltpu.VMEM((B, H), f32),
            pltpu.VMEM((B, H), f32),
            pltpu.VMEM((B, H), f32),
            pltpu.VMEM((B, 1), jnp.int32),
            pltpu.VMEM((B, 1), jnp.int32),
            pltpu.VMEM((B, 1), jnp.int32),
            pltpu.VMEM((B, NPT), f32),
            pltpu.VMEM((B, NPT), f32),
        ],
        compiler_params=pltpu.CompilerParams(
            dimension_semantics=("arbitrary", "arbitrary")),
    )(z, wi1, wi2c, wi2h, wih, whh, wtype, wn, wdt, wdn, embs, gt, gk)

    out_type, out_node = pl.pallas_call(
        _onehot_body,
        grid=(T,),
        in_specs=[
            pl.BlockSpec((1, B, 1), lambda i: (i, 0, 0)),
            pl.BlockSpec((1, B, 1), lambda i: (i, 0, 0)),
        ],
        out_specs=[
            pl.BlockSpec((1, B, NC), lambda i: (i, 0, 0)),
            pl.BlockSpec((1, B, NN + 1), lambda i: (i, 0, 0)),
        ],
        out_shape=[
            jax.ShapeDtypeStruct((T, B, NC), f32),
            jax.ShapeDtypeStruct((T, B, NN + 1), f32),
        ],
        compiler_params=pltpu.CompilerParams(
            dimension_semantics=("arbitrary",)),
    )(x_ids, node_ids)
    return out_type.transpose(1, 0, 2), out_node.transpose(1, 0, 2)
